# Initial kernel scaffold; baseline (speedup 1.0000x reference)
import jax
import jax.numpy as jnp
from jax.experimental import pallas as pl


def kernel(params, x, edge_index, batch):
    raise NotImplementedError("write your pallas kernel here")


# speedup vs baseline: 1.0011x; 1.0011x over previous
import functools

import jax
import jax.numpy as jnp
from jax.experimental import pallas as pl
from jax.experimental.pallas import tpu as pltpu

N = 10000
D = 256
H = 3
C = 256
L = 4
G = 256

_BN = 400


def _pool_body(batch_ref, *refs):
    xs_refs = refs[:L + 1]
    out_ref = refs[L + 1]
    acc_ref, cnt_ref = refs[L + 2], refs[L + 3]
    j = pl.program_id(0)

    @pl.when(j == 0)
    def _():
        acc_ref[...] = jnp.zeros_like(acc_ref)
        cnt_ref[...] = jnp.zeros_like(cnt_ref)

    xsum = xs_refs[0][...]
    for r in xs_refs[1:]:
        xsum = xsum + r[...]
    b = batch_ref[0]
    g = jax.lax.broadcasted_iota(jnp.int32, (G, _BN), 0)
    a = (g == b).astype(jnp.float32)
    acc_ref[...] += jax.lax.dot(a, xsum, preferred_element_type=jnp.float32)
    cnt_ref[...] += jnp.sum(a, axis=1, keepdims=True)

    @pl.when(j == pl.num_programs(0) - 1)
    def _():
        denom = (L + 1) * jnp.maximum(cnt_ref[...], 1.0)
        out_ref[...] = acc_ref[...] / denom


def _pool(xs, batch):
    grid = N // _BN
    batch2 = batch.reshape(grid, 1, _BN)
    return pl.pallas_call(
        _pool_body,
        grid=(grid,),
        in_specs=[pl.BlockSpec((1, 1, _BN), lambda j: (j, 0, 0))]
        + [pl.BlockSpec((_BN, C), lambda j: (j, 0)) for _ in range(L + 1)],
        out_specs=pl.BlockSpec((G, C), lambda j: (0, 0)),
        out_shape=jax.ShapeDtypeStruct((G, C), jnp.float32),
        scratch_shapes=[
            pltpu.VMEM((G, C), jnp.float32),
            pltpu.VMEM((G, 1), jnp.float32),
        ],
    )(batch2, *xs)


def _layer_norm(x, g, b):
    m = x.mean(-1, keepdims=True)
    v = ((x - m) ** 2).mean(-1, keepdims=True)
    return (x - m) / jnp.sqrt(v + 1e-5) * g + b


def _gat_conv(x, src, dst, p):
    n = x.shape[0]
    h = (x @ p['W']).reshape(n, H, C)
    a_src = (h * p['att_src']).sum(-1)
    a_dst = (h * p['att_dst']).sum(-1)
    e = a_src[src] + a_dst[dst]
    e = jax.nn.leaky_relu(e, 0.2)
    e_max = jax.ops.segment_max(e, dst, num_segments=n)
    e = jnp.exp(e - e_max[dst])
    denom = jax.ops.segment_sum(e, dst, num_segments=n)
    alpha = e / (denom[dst] + 1e-16)
    msg = h[src] * alpha[:, :, None]
    out = jax.ops.segment_sum(msg, dst, num_segments=n)
    return out.mean(axis=1) + p['bias']


def kernel(params, x, edge_index, batch):
    tbl = params['emb'].at[0].set(0.0)
    xf = tbl[x].mean(axis=-2)
    src0, dst0 = edge_index[0], edge_index[1]
    loop = jnp.arange(xf.shape[0], dtype=src0.dtype)
    src = jnp.concatenate([src0, loop])
    dst = jnp.concatenate([dst0, loop])
    xs = [xf]
    for p in params['layers']:
        xf = _gat_conv(xf, src, dst, p)
        xf = _layer_norm(xf, p['ln_g'], p['ln_b'])
        xs.append(xf)
    return _pool(xs, batch)


# speedup vs baseline: 11.3415x; 11.3294x over previous
import dataclasses
import functools

import jax
import jax.numpy as jnp
from jax import lax
from jax.experimental import pallas as pl
from jax.experimental.pallas import tpu as pltpu
from jax.experimental.pallas import tpu_sc as plsc

N = 10000
E = 160000
D = 256
H = 3
C = 256
L = 4
G = 256

_NP = 10240
_HN = _NP // 2
_ET = E + N
_ECAP = 170496
_BN = 512
_DBN = 1024

_SC_MESH = functools.partial(
    plsc.VectorSubcoreMesh, core_axis_name="c", subcore_axis_name="s")

_SC_CP = pltpu.CompilerParams()
if "needs_layout_passes" in pltpu.CompilerParams.__dataclass_fields__:
    _SC_CP = dataclasses.replace(_SC_CP, needs_layout_passes=False)


_EMB_CB = 16
_EMB_IDX = _EMB_CB * 9


def _emb_body(tbl_hbm, idx_hbm, out_hbm, idx_v, rows_v, obuf):
    c = lax.axis_index("c")
    s = lax.axis_index("s")
    wid = c * 16 + s
    npw = _NP // 32
    base_n = wid * npw

    @pl.loop(0, npw // _EMB_CB)
    def _(k):
        nb = base_n + k * _EMB_CB
        pltpu.sync_copy(idx_hbm.at[pl.ds(nb * 9, _EMB_IDX)], idx_v)
        pltpu.sync_copy(tbl_hbm.at[idx_v], rows_v)

        @pl.loop(0, _EMB_CB)
        def _(j):
            for q in range(D // 16):
                acc = rows_v[j * 9, pl.ds(q * 16, 16)]
                for r in range(1, 9):
                    acc = acc + rows_v[j * 9 + r, pl.ds(q * 16, 16)]
                obuf[j, pl.ds(q * 16, 16)] = acc * jnp.float32(1.0 / 9.0)

        pltpu.sync_copy(obuf, out_hbm.at[pl.ds(nb, _EMB_CB)])


def _embed(tbl, idx_flat):
    k = pl.kernel(
        _emb_body,
        out_type=jax.ShapeDtypeStruct((_NP, D), jnp.float32),
        mesh=_SC_MESH(),
        scratch_types=[
            pltpu.VMEM((_EMB_IDX,), jnp.int32),
            pltpu.VMEM((_EMB_IDX, D), jnp.float32),
            pltpu.VMEM((_EMB_CB, D), jnp.float32),
        ],
    )
    return k(tbl, idx_flat)


def _dense0_body(w_ref, attc_ref, x_ref, h_ref, a_ref):
    h = jnp.dot(x_ref[...], w_ref[...], preferred_element_type=jnp.float32)
    h_ref[...] = h
    a_ref[...] = jnp.dot(h, attc_ref[...], preferred_element_type=jnp.float32)


def _dense0(x, w, attc):
    grid = _NP // _DBN
    return pl.pallas_call(
        _dense0_body,
        grid=(grid,),
        in_specs=[
            pl.BlockSpec((D, H * C), lambda j: (0, 0)),
            pl.BlockSpec((H * C, 32), lambda j: (0, 0)),
            pl.BlockSpec((_DBN, D), lambda j: (j, 0)),
        ],
        out_specs=[
            pl.BlockSpec((_DBN, H * C), lambda j: (j, 0)),
            pl.BlockSpec((_DBN, 32), lambda j: (j, 0)),
        ],
        out_shape=[
            jax.ShapeDtypeStruct((_NP, H * C), jnp.float32),
            jax.ShapeDtypeStruct((_NP, 32), jnp.float32),
        ],
    )(w, attc, x)


def _ln_block(r, lng_ref, lnb_ref):
    m = jnp.mean(r, axis=-1, keepdims=True)
    v = jnp.mean((r - m) ** 2, axis=-1, keepdims=True)
    return (r - m) * lax.rsqrt(v + 1e-5) * lng_ref[...] + lnb_ref[...]


def _dense_body(bias_ref, lng_ref, lnb_ref, w_ref, attc_ref, raw_ref,
                x_ref, h_ref, a_ref):
    xl = _ln_block(raw_ref[...] + bias_ref[...], lng_ref, lnb_ref)
    x_ref[...] = xl
    h = jnp.dot(xl, w_ref[...], preferred_element_type=jnp.float32)
    h_ref[...] = h
    a_ref[...] = jnp.dot(h, attc_ref[...], preferred_element_type=jnp.float32)


def _dense(raw, bias, lng, lnb, w, attc):
    grid = _NP // _DBN
    return pl.pallas_call(
        _dense_body,
        grid=(grid,),
        in_specs=[
            pl.BlockSpec((1, D), lambda j: (0, 0)),
            pl.BlockSpec((1, D), lambda j: (0, 0)),
            pl.BlockSpec((1, D), lambda j: (0, 0)),
            pl.BlockSpec((D, H * C), lambda j: (0, 0)),
            pl.BlockSpec((H * C, 32), lambda j: (0, 0)),
            pl.BlockSpec((_DBN, D), lambda j: (j, 0)),
        ],
        out_specs=[
            pl.BlockSpec((_DBN, D), lambda j: (j, 0)),
            pl.BlockSpec((_DBN, H * C), lambda j: (j, 0)),
            pl.BlockSpec((_DBN, 32), lambda j: (j, 0)),
        ],
        out_shape=[
            jax.ShapeDtypeStruct((_NP, D), jnp.float32),
            jax.ShapeDtypeStruct((_NP, H * C), jnp.float32),
            jax.ShapeDtypeStruct((_NP, 32), jnp.float32),
        ],
    )(bias.reshape(1, D), lng.reshape(1, D), lnb.reshape(1, D), w, attc, raw)


def _postln_body(bias_ref, lng_ref, lnb_ref, raw_ref, x_ref):
    x_ref[...] = _ln_block(raw_ref[...] + bias_ref[...], lng_ref, lnb_ref)


def _postln(raw, bias, lng, lnb):
    grid = _NP // _DBN
    return pl.pallas_call(
        _postln_body,
        grid=(grid,),
        in_specs=[
            pl.BlockSpec((1, D), lambda j: (0, 0)),
            pl.BlockSpec((1, D), lambda j: (0, 0)),
            pl.BlockSpec((1, D), lambda j: (0, 0)),
            pl.BlockSpec((_DBN, D), lambda j: (j, 0)),
        ],
        out_specs=pl.BlockSpec((_DBN, D), lambda j: (j, 0)),
        out_shape=jax.ShapeDtypeStruct((_NP, D), jnp.float32),
    )(bias.reshape(1, D), lng.reshape(1, D), lnb.reshape(1, D), raw)


_ECH = 32


_STRIPE = _HN // 16
_EB = 128
_NDB = _HN // 128


_NT = 32
_NPT = _NP // _NT
_DUM = _NPT


def _edge_body(h_hbm, as_hbm, ad_hbm, srcg_hbm, dstg_hbm, ebnd_hbm, out_hbm,
               sidx, dsg, iref, asb0, asb1, asb2, adb0, adb1, adb2,
               wbuf, hb, eb, out_t, den0, den1, den2):
    c = lax.axis_index("c")
    s = lax.axis_index("s")
    wid = c * 16 + s
    nlo = wid * _NPT
    lane = lax.iota(jnp.int32, 16)
    lane16 = lane * 16
    asbs = (asb0, asb1, asb2)
    adbs = (adb0, adb1, adb2)
    dens = (den0, den1, den2)

    pltpu.sync_copy(ebnd_hbm, eb)
    sm = (lane == s).astype(jnp.int32)
    elo = jnp.sum(eb[pl.ds(c * 16, 16)] * sm)
    ehi = jnp.sum(eb[pl.ds(32 + c * 16, 16)] * sm)
    blo = elo // _EB
    bhi = (ehi + _EB - 1) // _EB

    zv = jnp.zeros((16,), jnp.float32)

    @pl.loop(0, _NPT + 8)
    def _(j):
        jv = lane * 0 + j
        for q in range(D // 16):
            plsc.store_scatter(out_t, [jv, lane + q * 16], zv)

    for r in range((_NPT + 64) // 16):
        for hh in range(H):
            plsc.store_scatter(dens[hh], [lane + r * 16], zv)

    def _load_block(base):
        pltpu.sync_copy(srcg_hbm.at[pl.ds(base, _EB)], sidx)
        pltpu.sync_copy(dstg_hbm.at[pl.ds(base, _EB)], dsg)
        for hh in range(H):
            for g in range(_EB // 16):
                iref[pl.ds(g * 16, 16)] = sidx[pl.ds(g * 16, 16)] + hh * _NP
            pltpu.sync_copy(as_hbm.at[iref], asbs[hh])
            for g in range(_EB // 16):
                iref[pl.ds(g * 16, 16)] = dsg[pl.ds(g * 16, 16)] + hh * _NP
            pltpu.sync_copy(ad_hbm.at[iref], adbs[hh])

    def _group(g):
        lrel = dsg[pl.ds(g * 16, 16)] - nlo
        mine = (lrel >= 0) & (lrel < _NPT)
        okf = mine.astype(jnp.float32)
        lcl = jnp.where(mine, lrel, _DUM)
        return okf, lcl

    def _escore(hh, g, okf):
        e = asbs[hh][pl.ds(g * 16, 16)] + adbs[hh][pl.ds(g * 16, 16)]
        e = jnp.maximum(e, jnp.float32(0.2) * e)
        return jnp.exp(e) * okf

    @pl.loop(blo, bhi)
    def _(blk):
        base = blk * _EB
        _load_block(base)
        for g in range(_EB // 16):
            okf, lcl = _group(g)
            for hh in range(H):
                plsc.addupdate_scatter(dens[hh], [lcl],
                                       _escore(hh, g, okf))

    for r in range((_NPT + 64) // 16):
        for hh in range(H):
            d = plsc.load_gather(dens[hh], [lane + r * 16])
            plsc.store_scatter(
                dens[hh], [lane + r * 16],
                jnp.float32(1.0 / 3.0) / (d + jnp.float32(1e-16)))

    m0 = (lane == 0).astype(jnp.float32)
    m1 = (lane == 1).astype(jnp.float32)
    m2 = (lane == 2).astype(jnp.float32)

    @pl.loop(blo, bhi)
    def _(blk):
        base = blk * _EB
        _load_block(base)
        for g in range(_EB // 16):
            okf, lcl = _group(g)
            ssm = sidx[pl.ds(g * 16, 16)]
            pltpu.sync_copy(h_hbm.at[ssm], hb)
            for hh in range(H):
                w = _escore(hh, g, okf) * plsc.load_gather(dens[hh], [lcl])
                plsc.store_scatter(wbuf, [lane16 + hh], w)
            lclf = lcl.astype(jnp.float32)

            @pl.loop(0, 16)
            def _(j):
                jh = lane * 0 + j
                mj = (lane == j).astype(jnp.float32)
                wrow = plsc.load_gather(wbuf, [lane + j * 16])
                w0 = jnp.sum(wrow * m0)
                w1 = jnp.sum(wrow * m1)
                w2 = jnp.sum(wrow * m2)
                lj = jnp.sum(lclf * mj).astype(jnp.int32)
                jrel = lane * 0 + lj
                for q in range(D // 16):
                    qv = lane + q * 16
                    acc = (plsc.load_gather(hb, [jh, qv]) * w0
                           + plsc.load_gather(hb, [jh, qv + C]) * w1
                           + plsc.load_gather(hb, [jh, qv + 2 * C]) * w2)
                    plsc.addupdate_scatter(out_t, [jrel, qv], acc)

    pltpu.sync_copy(out_t.at[pl.ds(0, _NPT)], out_hbm.at[pl.ds(nlo, _NPT)])


def _edge(h, asrcT, adstT, srcg, dstg, ebnd):
    k = pl.kernel(
        _edge_body,
        out_type=jax.ShapeDtypeStruct((_NP, D), jnp.float32),
        mesh=_SC_MESH(),
        scratch_types=[
            pltpu.VMEM((_EB,), jnp.int32),
            pltpu.VMEM((_EB,), jnp.int32),
            pltpu.VMEM((_EB,), jnp.int32),
        ] + [pltpu.VMEM((_EB,), jnp.float32)] * 6
        + [
            pltpu.VMEM((256,), jnp.float32),
            pltpu.VMEM((16, H * C), jnp.float32),
            pltpu.VMEM((128,), jnp.int32),
            pltpu.VMEM((_NPT + 8, D), jnp.float32),
        ] + [pltpu.VMEM((_NPT + 64,), jnp.float32)] * 3,
        compiler_params=_SC_CP,
    )
    return k(h, asrcT, adstT, srcg, dstg, ebnd)


def _pool_body(batch_ref, *refs):
    xs_refs = refs[:L + 1]
    out_ref = refs[L + 1]
    acc_ref, cnt_ref = refs[L + 2], refs[L + 3]
    j = pl.program_id(0)

    @pl.when(j == 0)
    def _():
        acc_ref[...] = jnp.zeros_like(acc_ref)
        cnt_ref[...] = jnp.zeros_like(cnt_ref)

    xsum = xs_refs[0][...]
    for r in xs_refs[1:]:
        xsum = xsum + r[...]
    b = batch_ref[0]
    g = lax.broadcasted_iota(jnp.int32, (G, _BN), 0)
    a = (g == b).astype(jnp.float32)
    acc_ref[...] += lax.dot(a, xsum, preferred_element_type=jnp.float32)
    cnt_ref[...] += jnp.sum(a, axis=1, keepdims=True)

    @pl.when(j == pl.num_programs(0) - 1)
    def _():
        denom = (L + 1) * jnp.maximum(cnt_ref[...], 1.0)
        out_ref[...] = acc_ref[...] / denom


def _pool(xs, batch_pad):
    grid = _NP // _BN
    batch2 = batch_pad.reshape(grid, 1, _BN)
    return pl.pallas_call(
        _pool_body,
        grid=(grid,),
        in_specs=[pl.BlockSpec((1, 1, _BN), lambda j: (j, 0, 0))]
        + [pl.BlockSpec((_BN, C), lambda j: (j, 0)) for _ in range(L + 1)],
        out_specs=pl.BlockSpec((G, C), lambda j: (0, 0)),
        out_shape=jax.ShapeDtypeStruct((G, C), jnp.float32),
        scratch_shapes=[
            pltpu.VMEM((G, C), jnp.float32),
            pltpu.VMEM((G, 1), jnp.float32),
        ],
    )(batch2, *xs)


def _pack_att(p):
    attc = jnp.zeros((H * C, 32), jnp.float32)
    for hh in range(H):
        attc = attc.at[hh * C:(hh + 1) * C, hh].set(p['att_src'][hh])
        attc = attc.at[hh * C:(hh + 1) * C, 16 + hh].set(p['att_dst'][hh])
    return attc


def kernel(params, x, edge_index, batch):
    tbl = params['emb'].at[0].set(0.0)
    idx_flat = jnp.zeros((_NP * 9,), jnp.int32).at[:N * 9].set(x.reshape(-1))
    x0 = _embed(tbl, idx_flat)

    loop = jnp.arange(N, dtype=jnp.int32)
    src = jnp.concatenate([edge_index[0], loop])
    dst = jnp.concatenate([edge_index[1], loop])
    order = jnp.argsort(dst)
    src_s = src[order]
    dst_s = dst[order]
    pad = _ECAP - _ET
    srcg = jnp.concatenate([src_s, jnp.zeros((pad,), jnp.int32)])
    dstg = jnp.concatenate([dst_s, jnp.full((pad,), -1, jnp.int32)])
    cuts = jnp.arange(33, dtype=jnp.int32) * (_NP // 32)
    bnd = jnp.searchsorted(dst_s, cuts).astype(jnp.int32)
    ebnd = jnp.zeros((128,), jnp.int32)
    ebnd = ebnd.at[0:32].set(bnd[0:32]).at[32:64].set(bnd[1:33])

    xs = [x0]
    xcur = x0
    raw = None
    for l in range(L):
        p = params['layers'][l]
        attc = _pack_att(p)
        if l == 0:
            h, a = _dense0(x0, p['W'], attc)
        else:
            pprev = params['layers'][l - 1]
            xcur, h, a = _dense(raw, pprev['bias'], pprev['ln_g'],
                                pprev['ln_b'], p['W'], attc)
            xs.append(xcur)
        asrcT = a[:, :H].T.reshape(-1)
        adstT = a[:, 16:16 + H].T.reshape(-1)
        raw = _edge(h, asrcT, adstT, srcg, dstg, ebnd)
    plast = params['layers'][L - 1]
    x4 = _postln(raw, plast['bias'], plast['ln_g'], plast['ln_b'])
    xs.append(x4)

    batch_pad = jnp.concatenate(
        [batch.astype(jnp.int32), jnp.full((_NP - N,), G, jnp.int32)])
    return _pool(xs, batch_pad)


# speedup vs baseline: 13.5265x; 1.1927x over previous
import dataclasses
import functools

import jax
import jax.numpy as jnp
from jax import lax
from jax.experimental import pallas as pl
from jax.experimental.pallas import tpu as pltpu
from jax.experimental.pallas import tpu_sc as plsc

N = 10000
E = 160000
D = 256
H = 3
C = 256
L = 4
G = 256

_NP = 10240
_HN = _NP // 2
_ET = E + N
_ECAP = 170496
_BN = 512
_DBN = 1024

_SC_MESH = functools.partial(
    plsc.VectorSubcoreMesh, core_axis_name="c", subcore_axis_name="s")

_SC_CP = pltpu.CompilerParams()
if "needs_layout_passes" in pltpu.CompilerParams.__dataclass_fields__:
    _SC_CP = dataclasses.replace(_SC_CP, needs_layout_passes=False)


_EMB_CB = 16
_EMB_IDX = _EMB_CB * 9


def _emb_body(tbl_hbm, idx_hbm, out_hbm, idx_v, rows_v, obuf):
    c = lax.axis_index("c")
    s = lax.axis_index("s")
    wid = c * 16 + s
    npw = _NP // 32
    base_n = wid * npw

    @pl.loop(0, npw // _EMB_CB)
    def _(k):
        nb = base_n + k * _EMB_CB
        pltpu.sync_copy(idx_hbm.at[pl.ds(nb * 9, _EMB_IDX)], idx_v)
        pltpu.sync_copy(tbl_hbm.at[idx_v], rows_v)

        @pl.loop(0, _EMB_CB)
        def _(j):
            for q in range(D // 16):
                acc = rows_v[j * 9, pl.ds(q * 16, 16)]
                for r in range(1, 9):
                    acc = acc + rows_v[j * 9 + r, pl.ds(q * 16, 16)]
                obuf[j, pl.ds(q * 16, 16)] = acc * jnp.float32(1.0 / 9.0)

        pltpu.sync_copy(obuf, out_hbm.at[pl.ds(nb, _EMB_CB)])


def _embed(tbl, idx_flat):
    k = pl.kernel(
        _emb_body,
        out_type=jax.ShapeDtypeStruct((_NP, D), jnp.float32),
        mesh=_SC_MESH(),
        scratch_types=[
            pltpu.VMEM((_EMB_IDX,), jnp.int32),
            pltpu.VMEM((_EMB_IDX, D), jnp.float32),
            pltpu.VMEM((_EMB_CB, D), jnp.float32),
        ],
    )
    return k(tbl, idx_flat)


def _dense0_body(w_ref, attc_ref, x_ref, h_ref, a_ref):
    h = jnp.dot(x_ref[...], w_ref[...], preferred_element_type=jnp.float32)
    h_ref[...] = h
    a_ref[...] = jnp.dot(h, attc_ref[...], preferred_element_type=jnp.float32)


def _dense0(x, w, attc):
    grid = _NP // _DBN
    return pl.pallas_call(
        _dense0_body,
        grid=(grid,),
        in_specs=[
            pl.BlockSpec((D, H * C), lambda j: (0, 0)),
            pl.BlockSpec((H * C, 32), lambda j: (0, 0)),
            pl.BlockSpec((_DBN, D), lambda j: (j, 0)),
        ],
        out_specs=[
            pl.BlockSpec((_DBN, H * C), lambda j: (j, 0)),
            pl.BlockSpec((_DBN, 32), lambda j: (j, 0)),
        ],
        out_shape=[
            jax.ShapeDtypeStruct((_NP, H * C), jnp.float32),
            jax.ShapeDtypeStruct((_NP, 32), jnp.float32),
        ],
    )(w, attc, x)


def _ln_block(r, lng_ref, lnb_ref):
    m = jnp.mean(r, axis=-1, keepdims=True)
    v = jnp.mean((r - m) ** 2, axis=-1, keepdims=True)
    return (r - m) * lax.rsqrt(v + 1e-5) * lng_ref[...] + lnb_ref[...]


def _dense_body(bias_ref, lng_ref, lnb_ref, w_ref, attc_ref, raw_ref,
                x_ref, h_ref, a_ref):
    xl = _ln_block(raw_ref[...] + bias_ref[...], lng_ref, lnb_ref)
    x_ref[...] = xl
    h = jnp.dot(xl, w_ref[...], preferred_element_type=jnp.float32)
    h_ref[...] = h
    a_ref[...] = jnp.dot(h, attc_ref[...], preferred_element_type=jnp.float32)


def _dense(raw, bias, lng, lnb, w, attc):
    grid = _NP // _DBN
    return pl.pallas_call(
        _dense_body,
        grid=(grid,),
        in_specs=[
            pl.BlockSpec((1, D), lambda j: (0, 0)),
            pl.BlockSpec((1, D), lambda j: (0, 0)),
            pl.BlockSpec((1, D), lambda j: (0, 0)),
            pl.BlockSpec((D, H * C), lambda j: (0, 0)),
            pl.BlockSpec((H * C, 32), lambda j: (0, 0)),
            pl.BlockSpec((_DBN, D), lambda j: (j, 0)),
        ],
        out_specs=[
            pl.BlockSpec((_DBN, D), lambda j: (j, 0)),
            pl.BlockSpec((_DBN, H * C), lambda j: (j, 0)),
            pl.BlockSpec((_DBN, 32), lambda j: (j, 0)),
        ],
        out_shape=[
            jax.ShapeDtypeStruct((_NP, D), jnp.float32),
            jax.ShapeDtypeStruct((_NP, H * C), jnp.float32),
            jax.ShapeDtypeStruct((_NP, 32), jnp.float32),
        ],
    )(bias.reshape(1, D), lng.reshape(1, D), lnb.reshape(1, D), w, attc, raw)


def _postln_body(bias_ref, lng_ref, lnb_ref, raw_ref, x_ref):
    x_ref[...] = _ln_block(raw_ref[...] + bias_ref[...], lng_ref, lnb_ref)


def _postln(raw, bias, lng, lnb):
    grid = _NP // _DBN
    return pl.pallas_call(
        _postln_body,
        grid=(grid,),
        in_specs=[
            pl.BlockSpec((1, D), lambda j: (0, 0)),
            pl.BlockSpec((1, D), lambda j: (0, 0)),
            pl.BlockSpec((1, D), lambda j: (0, 0)),
            pl.BlockSpec((_DBN, D), lambda j: (j, 0)),
        ],
        out_specs=pl.BlockSpec((_DBN, D), lambda j: (j, 0)),
        out_shape=jax.ShapeDtypeStruct((_NP, D), jnp.float32),
    )(bias.reshape(1, D), lng.reshape(1, D), lnb.reshape(1, D), raw)


_ECH = 32


_STRIPE = _HN // 16
_EB = 128
_NDB = _HN // 128


_NT = 32
_NPT = _NP // _NT
_DUM = _NPT


def _edge_body(h_hbm, as_hbm, ad_hbm, srcg_hbm, dstg_hbm, ebnd_hbm, out_hbm,
               sidx, dsg, iref, asb0, asb1, asb2, adb0, adb1, adb2,
               wbuf, hb, hb2, eb, out_t, den0, den1, den2, sem0, sem1):
    c = lax.axis_index("c")
    s = lax.axis_index("s")
    wid = c * 16 + s
    nlo = wid * _NPT
    lane = lax.iota(jnp.int32, 16)
    lane16 = lane * 16
    asbs = (asb0, asb1, asb2)
    adbs = (adb0, adb1, adb2)
    dens = (den0, den1, den2)

    pltpu.sync_copy(ebnd_hbm, eb)
    sm = (lane == s).astype(jnp.int32)
    elo = jnp.sum(eb[pl.ds(c * 16, 16)] * sm)
    ehi = jnp.sum(eb[pl.ds(32 + c * 16, 16)] * sm)
    blo = elo // _EB
    bhi = (ehi + _EB - 1) // _EB

    zv = jnp.zeros((16,), jnp.float32)

    @pl.loop(0, _NPT + 8)
    def _(j):
        jv = lane * 0 + j
        for q in range(D // 16):
            plsc.store_scatter(out_t, [jv, lane + q * 16], zv)

    for r in range((_NPT + 64) // 16):
        for hh in range(H):
            plsc.store_scatter(dens[hh], [lane + r * 16], zv)

    def _load_block(base):
        pltpu.sync_copy(srcg_hbm.at[pl.ds(base, _EB)], sidx)
        pltpu.sync_copy(dstg_hbm.at[pl.ds(base, _EB)], dsg)
        for hh in range(H):
            for g in range(_EB // 16):
                iref[pl.ds(g * 16, 16)] = sidx[pl.ds(g * 16, 16)] + hh * _NP
            pltpu.sync_copy(as_hbm.at[iref], asbs[hh])
            for g in range(_EB // 16):
                iref[pl.ds(g * 16, 16)] = dsg[pl.ds(g * 16, 16)] + hh * _NP
            pltpu.sync_copy(ad_hbm.at[iref], adbs[hh])

    def _group(g):
        lrel = dsg[pl.ds(g * 16, 16)] - nlo
        mine = (lrel >= 0) & (lrel < _NPT)
        okf = mine.astype(jnp.float32)
        lcl = jnp.where(mine, lrel, _DUM)
        return okf, lcl

    def _escore(hh, g, okf):
        e = asbs[hh][pl.ds(g * 16, 16)] + adbs[hh][pl.ds(g * 16, 16)]
        e = jnp.maximum(e, jnp.float32(0.2) * e)
        return jnp.exp(e) * okf

    @pl.loop(blo, bhi)
    def _(blk):
        base = blk * _EB
        _load_block(base)
        for g in range(_EB // 16):
            okf, lcl = _group(g)
            for hh in range(H):
                plsc.addupdate_scatter(dens[hh], [lcl],
                                       _escore(hh, g, okf))

    for r in range((_NPT + 64) // 16):
        for hh in range(H):
            d = plsc.load_gather(dens[hh], [lane + r * 16])
            plsc.store_scatter(
                dens[hh], [lane + r * 16],
                jnp.float32(1.0 / 3.0) / (d + jnp.float32(1e-16)))

    m0 = (lane == 0).astype(jnp.float32)
    m1 = (lane == 1).astype(jnp.float32)
    m2 = (lane == 2).astype(jnp.float32)

    @pl.loop(blo, bhi)
    def _(blk):
        base = blk * _EB
        _load_block(base)
        hbufs = (hb, hb2)
        sems = (sem0, sem1)
        cp = pltpu.make_async_copy(h_hbm.at[sidx[pl.ds(0, 16)]], hb, sem0)
        cp.start()
        cps = {0: cp}
        for g in range(_EB // 16):
            okf, lcl = _group(g)
            if g < _EB // 16 - 1:
                nxt = pltpu.make_async_copy(
                    h_hbm.at[sidx[pl.ds((g + 1) * 16, 16)]],
                    hbufs[(g + 1) % 2], sems[(g + 1) % 2])
                nxt.start()
                cps[g + 1] = nxt
            cps[g].wait()
            hcur = hbufs[g % 2]
            for hh in range(H):
                w = _escore(hh, g, okf) * plsc.load_gather(dens[hh], [lcl])
                plsc.store_scatter(wbuf, [lane16 + hh], w)
            lclf = lcl.astype(jnp.float32)

            @pl.loop(0, 16)
            def _(j, hcur=hcur):
                jh = lane * 0 + j
                mj = (lane == j).astype(jnp.float32)
                wrow = plsc.load_gather(wbuf, [lane + j * 16])
                w0 = jnp.sum(wrow * m0)
                w1 = jnp.sum(wrow * m1)
                w2 = jnp.sum(wrow * m2)
                lj = jnp.sum(lclf * mj).astype(jnp.int32)
                jrel = lane * 0 + lj
                for q in range(D // 16):
                    qv = lane + q * 16
                    acc = (plsc.load_gather(hcur, [jh, qv]) * w0
                           + plsc.load_gather(hcur, [jh, qv + C]) * w1
                           + plsc.load_gather(hcur, [jh, qv + 2 * C]) * w2)
                    plsc.addupdate_scatter(out_t, [jrel, qv], acc)

    pltpu.sync_copy(out_t.at[pl.ds(0, _NPT)], out_hbm.at[pl.ds(nlo, _NPT)])


def _edge(h, asrcT, adstT, srcg, dstg, ebnd):
    k = pl.kernel(
        _edge_body,
        out_type=jax.ShapeDtypeStruct((_NP, D), jnp.float32),
        mesh=_SC_MESH(),
        scratch_types=[
            pltpu.VMEM((_EB,), jnp.int32),
            pltpu.VMEM((_EB,), jnp.int32),
            pltpu.VMEM((_EB,), jnp.int32),
        ] + [pltpu.VMEM((_EB,), jnp.float32)] * 6
        + [
            pltpu.VMEM((256,), jnp.float32),
            pltpu.VMEM((16, H * C), jnp.float32),
            pltpu.VMEM((16, H * C), jnp.float32),
            pltpu.VMEM((128,), jnp.int32),
            pltpu.VMEM((_NPT + 8, D), jnp.float32),
        ] + [pltpu.VMEM((_NPT + 64,), jnp.float32)] * 3
        + [pltpu.SemaphoreType.DMA] * 2,
        compiler_params=_SC_CP,
    )
    return k(h, asrcT, adstT, srcg, dstg, ebnd)


def _pool_body(batch_ref, *refs):
    xs_refs = refs[:L + 1]
    out_ref = refs[L + 1]
    acc_ref, cnt_ref = refs[L + 2], refs[L + 3]
    j = pl.program_id(0)

    @pl.when(j == 0)
    def _():
        acc_ref[...] = jnp.zeros_like(acc_ref)
        cnt_ref[...] = jnp.zeros_like(cnt_ref)

    xsum = xs_refs[0][...]
    for r in xs_refs[1:]:
        xsum = xsum + r[...]
    b = batch_ref[0]
    g = lax.broadcasted_iota(jnp.int32, (G, _BN), 0)
    a = (g == b).astype(jnp.float32)
    acc_ref[...] += lax.dot(a, xsum, preferred_element_type=jnp.float32)
    cnt_ref[...] += jnp.sum(a, axis=1, keepdims=True)

    @pl.when(j == pl.num_programs(0) - 1)
    def _():
        denom = (L + 1) * jnp.maximum(cnt_ref[...], 1.0)
        out_ref[...] = acc_ref[...] / denom


def _pool(xs, batch_pad):
    grid = _NP // _BN
    batch2 = batch_pad.reshape(grid, 1, _BN)
    return pl.pallas_call(
        _pool_body,
        grid=(grid,),
        in_specs=[pl.BlockSpec((1, 1, _BN), lambda j: (j, 0, 0))]
        + [pl.BlockSpec((_BN, C), lambda j: (j, 0)) for _ in range(L + 1)],
        out_specs=pl.BlockSpec((G, C), lambda j: (0, 0)),
        out_shape=jax.ShapeDtypeStruct((G, C), jnp.float32),
        scratch_shapes=[
            pltpu.VMEM((G, C), jnp.float32),
            pltpu.VMEM((G, 1), jnp.float32),
        ],
    )(batch2, *xs)


def _pack_att(p):
    attc = jnp.zeros((H * C, 32), jnp.float32)
    for hh in range(H):
        attc = attc.at[hh * C:(hh + 1) * C, hh].set(p['att_src'][hh])
        attc = attc.at[hh * C:(hh + 1) * C, 16 + hh].set(p['att_dst'][hh])
    return attc


def kernel(params, x, edge_index, batch):
    tbl = params['emb'].at[0].set(0.0)
    idx_flat = jnp.zeros((_NP * 9,), jnp.int32).at[:N * 9].set(x.reshape(-1))
    x0 = _embed(tbl, idx_flat)

    loop = jnp.arange(N, dtype=jnp.int32)
    src = jnp.concatenate([edge_index[0], loop])
    dst = jnp.concatenate([edge_index[1], loop])
    order = jnp.argsort(dst)
    src_s = src[order]
    dst_s = dst[order]
    pad = _ECAP - _ET
    srcg = jnp.concatenate([src_s, jnp.zeros((pad,), jnp.int32)])
    dstg = jnp.concatenate([dst_s, jnp.full((pad,), -1, jnp.int32)])
    cuts = jnp.arange(33, dtype=jnp.int32) * (_NP // 32)
    bnd = jnp.searchsorted(dst_s, cuts).astype(jnp.int32)
    ebnd = jnp.zeros((128,), jnp.int32)
    ebnd = ebnd.at[0:32].set(bnd[0:32]).at[32:64].set(bnd[1:33])

    xs = [x0]
    xcur = x0
    raw = None
    for l in range(L):
        p = params['layers'][l]
        attc = _pack_att(p)
        if l == 0:
            h, a = _dense0(x0, p['W'], attc)
        else:
            pprev = params['layers'][l - 1]
            xcur, h, a = _dense(raw, pprev['bias'], pprev['ln_g'],
                                pprev['ln_b'], p['W'], attc)
            xs.append(xcur)
        asrcT = a[:, :H].T.reshape(-1)
        adstT = a[:, 16:16 + H].T.reshape(-1)
        raw = _edge(h, asrcT, adstT, srcg, dstg, ebnd)
    plast = params['layers'][L - 1]
    x4 = _postln(raw, plast['bias'], plast['ln_g'], plast['ln_b'])
    xs.append(x4)

    batch_pad = jnp.concatenate(
        [batch.astype(jnp.int32), jnp.full((_NP - N,), G, jnp.int32)])
    return _pool(xs, batch_pad)
